# baseline (speedup 1.0000x reference)
import jax
import jax.numpy as jnp
from jax.experimental import pallas as pl
from jax.experimental.pallas import tpu as pltpu

T = 32768
V = 1024
BLANK = V - 1
BT = 512
NBLK = T // BT


def _body(x_ref, idx_ref, keep_ref, score_ref, prev_ref):
    i = pl.program_id(0)

    @pl.when(i == 0)
    def _():
        prev_ref[0] = -1

    v = x_ref[:, 0:128]
    bestc = jnp.zeros((BT, 128), jnp.int32)
    for c in range(1, 8):
        u = x_ref[:, c * 128 : (c + 1) * 128]
        gt = u > v
        v = jnp.where(gt, u, v)
        bestc = jnp.where(gt, c, bestc)
    vT = v.T
    cT = bestc.T
    m = jnp.max(vT, axis=0)
    lane0 = jax.lax.broadcasted_iota(jnp.int32, (128, BT), 0)
    posT = cT * 128 + lane0
    cand = jnp.where(vT == m[None, :], posT, V)
    idx = jnp.min(cand, axis=0)
    prev_first = jnp.full((1,), prev_ref[0], dtype=jnp.int32)
    prev = jnp.concatenate([prev_first, idx[: BT - 1]])
    keep = (idx != BLANK) & (idx != prev)
    idx_ref[...] = idx
    keep_ref[...] = keep
    score_ref[...] = m
    prev_ref[0] = idx[BT - 1]


def kernel(emission):
    idx, keep, scores = pl.pallas_call(
        _body,
        grid=(NBLK,),
        in_specs=[pl.BlockSpec((BT, V), lambda i: (i, 0))],
        out_specs=[
            pl.BlockSpec((BT,), lambda i: (i,)),
            pl.BlockSpec((BT,), lambda i: (i,)),
            pl.BlockSpec((BT,), lambda i: (i,)),
        ],
        out_shape=[
            jax.ShapeDtypeStruct((T,), jnp.int32),
            jax.ShapeDtypeStruct((T,), jnp.bool_),
            jax.ShapeDtypeStruct((T,), jnp.float32),
        ],
        scratch_shapes=[pltpu.SMEM((1,), jnp.int32)],
    )(emission)
    return idx, keep, scores


# speedup vs baseline: 1.3288x; 1.3288x over previous
import jax
import jax.numpy as jnp
from jax.experimental import pallas as pl
from jax.experimental.pallas import tpu as pltpu

T = 32768
V = 1024
BLANK = V - 1
BT = 1024
NBLK = T // BT


def _body(x_ref, idx_ref, keep_ref, score_ref, prev_ref):
    i = pl.program_id(0)

    @pl.when(i == 0)
    def _():
        prev_ref[0] = -1

    v = x_ref[:, 0:128]
    bestc = jnp.zeros((BT, 128), jnp.int32)
    for c in range(1, 8):
        u = x_ref[:, c * 128 : (c + 1) * 128]
        gt = u > v
        v = jnp.where(gt, u, v)
        bestc = jnp.where(gt, c, bestc)
    vT = v.T
    cT = bestc.T
    m = jnp.max(vT, axis=0)
    lane0 = jax.lax.broadcasted_iota(jnp.int32, (128, BT), 0)
    posT = cT * 128 + lane0
    cand = jnp.where(vT == m[None, :], posT, V)
    idx = jnp.min(cand, axis=0)
    prev_first = jnp.full((1,), prev_ref[0], dtype=jnp.int32)
    prev = jnp.concatenate([prev_first, idx[: BT - 1]])
    keep = (idx != BLANK) & (idx != prev)
    idx_ref[...] = idx
    keep_ref[...] = keep
    score_ref[...] = m
    prev_ref[0] = idx[BT - 1]


def kernel(emission):
    idx, keep, scores = pl.pallas_call(
        _body,
        grid=(NBLK,),
        in_specs=[pl.BlockSpec((BT, V), lambda i: (i, 0))],
        out_specs=[
            pl.BlockSpec((BT,), lambda i: (i,)),
            pl.BlockSpec((BT,), lambda i: (i,)),
            pl.BlockSpec((BT,), lambda i: (i,)),
        ],
        out_shape=[
            jax.ShapeDtypeStruct((T,), jnp.int32),
            jax.ShapeDtypeStruct((T,), jnp.bool_),
            jax.ShapeDtypeStruct((T,), jnp.float32),
        ],
        scratch_shapes=[pltpu.SMEM((1,), jnp.int32)],
    )(emission)
    return idx, keep, scores


# speedup vs baseline: 1.5411x; 1.1598x over previous
import jax
import jax.numpy as jnp
from jax.experimental import pallas as pl
from jax.experimental.pallas import tpu as pltpu

T = 32768
V = 1024
BLANK = V - 1
BT = 2048
NBLK = T // BT


def _body(x_ref, idx_ref, keep_ref, score_ref, prev_ref):
    i = pl.program_id(0)

    @pl.when(i == 0)
    def _():
        prev_ref[0] = -1

    v = x_ref[:, 0:128]
    bestc = jnp.zeros((BT, 128), jnp.int32)
    for c in range(1, 8):
        u = x_ref[:, c * 128 : (c + 1) * 128]
        gt = u > v
        v = jnp.where(gt, u, v)
        bestc = jnp.where(gt, c, bestc)
    vT = v.T
    cT = bestc.T
    m = jnp.max(vT, axis=0)
    lane0 = jax.lax.broadcasted_iota(jnp.int32, (128, BT), 0)
    posT = cT * 128 + lane0
    cand = jnp.where(vT == m[None, :], posT, V)
    idx = jnp.min(cand, axis=0)
    prev_first = jnp.full((1,), prev_ref[0], dtype=jnp.int32)
    prev = jnp.concatenate([prev_first, idx[: BT - 1]])
    keep = (idx != BLANK) & (idx != prev)
    idx_ref[...] = idx
    keep_ref[...] = keep
    score_ref[...] = m
    prev_ref[0] = idx[BT - 1]


def kernel(emission):
    idx, keep, scores = pl.pallas_call(
        _body,
        grid=(NBLK,),
        in_specs=[pl.BlockSpec((BT, V), lambda i: (i, 0))],
        out_specs=[
            pl.BlockSpec((BT,), lambda i: (i,)),
            pl.BlockSpec((BT,), lambda i: (i,)),
            pl.BlockSpec((BT,), lambda i: (i,)),
        ],
        out_shape=[
            jax.ShapeDtypeStruct((T,), jnp.int32),
            jax.ShapeDtypeStruct((T,), jnp.bool_),
            jax.ShapeDtypeStruct((T,), jnp.float32),
        ],
        scratch_shapes=[pltpu.SMEM((1,), jnp.int32)],
    )(emission)
    return idx, keep, scores
